# Initial kernel scaffold; baseline (speedup 1.0000x reference)
import functools

import jax
import jax.numpy as jnp
from jax import lax
from jax.experimental import pallas as pl
from jax.experimental.pallas import tpu as pltpu
from jax.experimental.pallas import tpu_sc as plsc

NC = 2
NS = 16
LANES = 16
EB = 128


def _matmul_body(x_ref, w_ref, o_ref):
    o_ref[...] = jnp.dot(x_ref[...], w_ref[...],
                         preferred_element_type=jnp.float32)


def _matmul(x, W, block_rows=1000):
    n, d = x.shape
    o = W.shape[1]
    grid = n // block_rows
    return pl.pallas_call(
        _matmul_body,
        grid=(grid,),
        in_specs=[
            pl.BlockSpec((block_rows, d), lambda i: (i, 0)),
            pl.BlockSpec((d, o), lambda i: (0, 0)),
        ],
        out_specs=pl.BlockSpec((block_rows, o), lambda i: (i, 0)),
        out_shape=jax.ShapeDtypeStruct((n, o), jnp.float32),
    )(x, W)


def _combine_body(a_ref, b_ref, o_ref):
    o_ref[...] = jnp.maximum(a_ref[...] + b_ref[...], 0.0)


def _combine(a, b, block_rows=1000):
    n, o = a.shape
    grid = n // block_rows
    return pl.pallas_call(
        _combine_body,
        grid=(grid,),
        in_specs=[
            pl.BlockSpec((block_rows, o), lambda i: (i, 0)),
            pl.BlockSpec((block_rows, o), lambda i: (i, 0)),
        ],
        out_specs=pl.BlockSpec((block_rows, o), lambda i: (i, 0)),
        out_shape=jax.ShapeDtypeStruct((n, o), jnp.float32),
    )(a, b)


def _sc_aggregate(h, srcp, dstp, wp, zeros, bpw, n, o):
    mesh = plsc.VectorSubcoreMesh(core_axis_name="c", subcore_axis_name="s")
    rows_per_tile = n // NS

    @functools.partial(
        pl.kernel,
        out_type=jax.ShapeDtypeStruct((NC, n, o), jnp.float32),
        mesh=mesh,
        scratch_types=dict(
            src_v=pltpu.VMEM((EB,), jnp.int32),
            dst_v=pltpu.VMEM((EB,), jnp.int32),
            w_v=pltpu.VMEM((EB,), jnp.float32),
            rows_v=pltpu.VMEM((EB, o), jnp.float32),
            accum=pltpu.VMEM_SHARED((n, o), jnp.float32),
            sem=pltpu.SemaphoreType.DMA,
        ),
    )
    def k(h_hbm, src_hbm, dst_hbm, w_hbm, z_hbm, out_hbm,
          src_v, dst_v, w_v, rows_v, accum, sem):
        c = lax.axis_index("c")
        s = lax.axis_index("s")
        wid = s * NC + c

        r0 = s * rows_per_tile
        pltpu.sync_copy(z_hbm.at[pl.ds(r0, rows_per_tile)],
                        accum.at[pl.ds(r0, rows_per_tile)])
        plsc.subcore_barrier()

        def body(i, _):
            blk = wid * bpw + i
            pltpu.sync_copy(src_hbm.at[blk], src_v)
            pltpu.sync_copy(dst_hbm.at[blk], dst_v)
            pltpu.sync_copy(w_hbm.at[blk], w_v)
            pltpu.async_copy(h_hbm.at[src_v], rows_v, sem).wait()

            def scale_row(r, _):
                wspl = plsc.load_gather(
                    w_v, [jnp.full((LANES,), r, jnp.int32)])
                row = rows_v.at[r]
                for cc in range(o // LANES):
                    sl = pl.ds(cc * LANES, LANES)
                    row[sl] = row[sl] * wspl
                return 0

            lax.fori_loop(0, EB, scale_row, 0)
            pltpu.sync_copy(rows_v, accum.at[dst_v], add=True)
            return 0

        lax.fori_loop(0, bpw, body, 0)
        plsc.subcore_barrier()

        pltpu.sync_copy(accum.at[pl.ds(r0, rows_per_tile)],
                        out_hbm.at[c, pl.ds(r0, rows_per_tile)])

    return k(h, srcp, dstp, wp, zeros)


def kernel(x, edge_index, edge_weight, W):
    n, d = x.shape
    o = W.shape[1]
    e = edge_weight.shape[0]

    h = _matmul(x, W)

    nw = NC * NS
    bpw = -(-e // (nw * EB))
    ep = nw * bpw * EB
    pad = ep - e
    src = jnp.concatenate([edge_index[0], jnp.zeros((pad,), jnp.int32)])
    dst = jnp.concatenate([edge_index[1], jnp.zeros((pad,), jnp.int32)])
    ew = jnp.concatenate([edge_weight, jnp.zeros((pad,), jnp.float32)])
    srcp = src.reshape(ep // EB, EB)
    dstp = dst.reshape(ep // EB, EB)
    wp = ew.reshape(ep // EB, EB)

    zeros = jnp.zeros((n, o), jnp.float32)
    partials = _sc_aggregate(h, srcp, dstp, wp, zeros, bpw, n, o)
    return _combine(partials[0], partials[1])


# speedup vs baseline: 3.8208x; 3.8208x over previous
import functools

import jax
import jax.numpy as jnp
from jax import lax
from jax.experimental import pallas as pl
from jax.experimental.pallas import tpu as pltpu
from jax.experimental.pallas import tpu_sc as plsc

NC = 2
NS = 16
LANES = 16
EB = 128


def _matmul_body(x_ref, w_ref, o_ref):
    o_ref[...] = jnp.dot(x_ref[...], w_ref[...],
                         preferred_element_type=jnp.float32)


def _matmul(x, W, block_rows=1000):
    n, d = x.shape
    o = W.shape[1]
    grid = n // block_rows
    return pl.pallas_call(
        _matmul_body,
        grid=(grid,),
        in_specs=[
            pl.BlockSpec((block_rows, d), lambda i: (i, 0)),
            pl.BlockSpec((d, o), lambda i: (0, 0)),
        ],
        out_specs=pl.BlockSpec((block_rows, o), lambda i: (i, 0)),
        out_shape=jax.ShapeDtypeStruct((n, o), jnp.float32),
    )(x, W)


def _combine_body(a_ref, b_ref, o_ref):
    o_ref[...] = jnp.maximum(a_ref[...] + b_ref[...], 0.0)


def _combine(a, b, block_rows=1000):
    n, o = a.shape
    grid = n // block_rows
    return pl.pallas_call(
        _combine_body,
        grid=(grid,),
        in_specs=[
            pl.BlockSpec((block_rows, o), lambda i: (i, 0)),
            pl.BlockSpec((block_rows, o), lambda i: (i, 0)),
        ],
        out_specs=pl.BlockSpec((block_rows, o), lambda i: (i, 0)),
        out_shape=jax.ShapeDtypeStruct((n, o), jnp.float32),
    )(a, b)


def _sc_aggregate(h, srcp, dstp, wp, zeros, bpw, n, o):
    mesh = plsc.VectorSubcoreMesh(core_axis_name="c", subcore_axis_name="s")
    rows_per_tile = n // NS

    @functools.partial(
        pl.kernel,
        out_type=jax.ShapeDtypeStruct((NC, n, o), jnp.float32),
        mesh=mesh,
        scratch_types=dict(
            src_v=pltpu.VMEM((EB,), jnp.int32),
            dst_v=pltpu.VMEM((EB,), jnp.int32),
            w_v=pltpu.VMEM((EB,), jnp.float32),
            rows_v=pltpu.VMEM((EB, o), jnp.float32),
            accum=pltpu.VMEM_SHARED((n, o), jnp.float32),
            sem=pltpu.SemaphoreType.DMA,
        ),
    )
    def k(h_hbm, src_hbm, dst_hbm, w_hbm, z_hbm, out_hbm,
          src_v, dst_v, w_v, rows_v, accum, sem):
        c = lax.axis_index("c")
        s = lax.axis_index("s")
        wid = s * NC + c

        r0 = s * rows_per_tile
        pltpu.sync_copy(z_hbm.at[pl.ds(r0, rows_per_tile)],
                        accum.at[pl.ds(r0, rows_per_tile)])
        plsc.subcore_barrier()

        def body(i, _):
            blk = wid * bpw + i
            pltpu.sync_copy(src_hbm.at[blk], src_v)
            pltpu.sync_copy(dst_hbm.at[blk], dst_v)
            pltpu.sync_copy(w_hbm.at[blk], w_v)
            pltpu.async_copy(h_hbm.at[src_v], rows_v, sem).wait()

            def scale_16rows(rb, _):
                w16 = w_v[pl.ds(rb * LANES, LANES)]
                for rr in range(LANES):
                    idx = jnp.full((LANES,), rr, jnp.int32)
                    wspl = w16.at[idx].get(mode="promise_in_bounds")
                    row = rows_v.at[rb * LANES + rr]
                    for cc in range(o // LANES):
                        sl = pl.ds(cc * LANES, LANES)
                        row[sl] = row[sl] * wspl
                return 0

            lax.fori_loop(0, EB // LANES, scale_16rows, 0)
            pltpu.sync_copy(rows_v, accum.at[dst_v], add=True)
            return 0

        lax.fori_loop(0, bpw, body, 0)
        plsc.subcore_barrier()

        pltpu.sync_copy(accum.at[pl.ds(r0, rows_per_tile)],
                        out_hbm.at[c, pl.ds(r0, rows_per_tile)])

    return k(h, srcp, dstp, wp, zeros)


def kernel(x, edge_index, edge_weight, W):
    n, d = x.shape
    o = W.shape[1]
    e = edge_weight.shape[0]

    h = _matmul(x, W)

    nw = NC * NS
    bpw = -(-e // (nw * EB))
    ep = nw * bpw * EB
    pad = ep - e
    src = jnp.concatenate([edge_index[0], jnp.zeros((pad,), jnp.int32)])
    dst = jnp.concatenate([edge_index[1], jnp.zeros((pad,), jnp.int32)])
    ew = jnp.concatenate([edge_weight, jnp.zeros((pad,), jnp.float32)])
    srcp = src.reshape(ep // EB, EB)
    dstp = dst.reshape(ep // EB, EB)
    wp = ew.reshape(ep // EB, EB)

    n_pad = -(-n // (NS * 8)) * NS * 8
    zeros = jnp.zeros((n_pad, o), jnp.float32)
    partials = _sc_aggregate(h, srcp, dstp, wp, zeros, bpw, n_pad, o)
    return _combine(partials[0, :n], partials[1, :n])
